# Initial kernel scaffold; baseline (speedup 1.0000x reference)
import jax
import jax.numpy as jnp
from jax.experimental import pallas as pl


def kernel(neighbor_feature, weight):
    raise NotImplementedError("write your pallas kernel here")


# speedup vs baseline: 1.1476x; 1.1476x over previous
import jax
import jax.numpy as jnp
from jax.experimental import pallas as pl
from jax.experimental.pallas import tpu as pltpu

_N, _K, _D = 10000, 32, 128
_R = 400


def _fused_body(nf_ref, w_ref, out_ref):
    agg = jnp.sum(nf_ref[...], axis=1) * (1.0 / _K)
    out_ref[...] = jnp.dot(agg, w_ref[...], preferred_element_type=jnp.float32)


def kernel(neighbor_feature, weight):
    grid = (_N // _R,)
    return pl.pallas_call(
        _fused_body,
        grid=grid,
        in_specs=[
            pl.BlockSpec((_R, _K, _D), lambda i: (i, 0, 0)),
            pl.BlockSpec((_D, _D), lambda i: (0, 0)),
        ],
        out_specs=pl.BlockSpec((_R, _D), lambda i: (i, 0)),
        out_shape=jax.ShapeDtypeStruct((_N, _D), jnp.float32),
    )(neighbor_feature, weight)
